# baseline (speedup 1.0000x reference)
import jax
import jax.numpy as jnp
from jax.experimental import pallas as pl
from jax.experimental.pallas import tpu as pltpu

_B, _S, _D = 16, 2048, 128
_C, _L = 256, 8
_EXT = 384


def _conv_body(cl_ref, x_ref, wt_ref, bias_ref, lt_ref, out_ref):
    x = x_ref[0]
    clf = cl_ref[0].astype(jnp.float32)
    csum = jnp.dot(clf, lt_ref[...], preferred_element_type=jnp.float32)
    csum_i = csum.astype(jnp.int32)
    ext = jnp.concatenate([csum_i, jnp.zeros((1, _EXT - _C), jnp.int32)], axis=1)
    s2 = jax.lax.broadcasted_iota(jnp.int32, (_S, _EXT), 0)
    e = jnp.broadcast_to(ext, (_S, _EXT))
    m = 1.0 - jnp.max((e == s2).astype(jnp.float32), axis=1, keepdims=True)
    total = jnp.max(ext)

    xl = jnp.concatenate([x[:1], x[:-1]], axis=0) * m
    xm = x * m
    xr = jnp.concatenate([xm[1:], xm[-1:]], axis=0)

    acc = jnp.dot(xl, wt_ref[0], preferred_element_type=jnp.float32)
    acc = acc + jnp.dot(x, wt_ref[1], preferred_element_type=jnp.float32)
    acc = acc + jnp.dot(xr, wt_ref[2], preferred_element_type=jnp.float32)
    acc = acc + bias_ref[0][None, :]
    sv = jax.lax.broadcasted_iota(jnp.int32, (_S, _D), 0)
    out_ref[0] = jnp.where(sv < total, jnp.maximum(acc, 0.0), 0.0)


def kernel(token_level_features, W, b, chunk_lens):
    x = token_level_features
    cl3 = chunk_lens.astype(jnp.int32).reshape(_B, 1, _C)
    wt = jnp.transpose(W, (2, 1, 0))
    bias2 = b.reshape(1, _D)
    lt = jnp.triu(jnp.ones((_C, _C), jnp.float32))

    out = pl.pallas_call(
        _conv_body,
        grid=(_B,),
        in_specs=[
            pl.BlockSpec((1, 1, _C), lambda i: (i, 0, 0)),
            pl.BlockSpec((1, _S, _D), lambda i: (i, 0, 0)),
            pl.BlockSpec((3, _D, _D), lambda i: (0, 0, 0)),
            pl.BlockSpec((1, _D), lambda i: (0, 0)),
            pl.BlockSpec((_C, _C), lambda i: (0, 0)),
        ],
        out_specs=pl.BlockSpec((1, _S, _D), lambda i: (i, 0, 0)),
        out_shape=jax.ShapeDtypeStruct((_B, _S, _D), x.dtype),
        compiler_params=pltpu.CompilerParams(
            dimension_semantics=("parallel",)),
    )(cl3, x, wt, bias2, lt)
    return out
